# baseline (speedup 1.0000x reference)
import functools

import jax
import jax.numpy as jnp
from jax import lax
from jax.experimental import pallas as pl
from jax.experimental.pallas import tpu as pltpu
from jax.experimental.pallas import tpu_sc as plsc

N = 10000
E = 160000
DIN = 256
H = 512
OUT = 128
G = 64

NC = 2
NS = 16
CW = 128
C = H // CW

NPAD = 10240
TN = 1024
NBN = NPAD // TN
ROWS_PER_TILE = NPAD // NS

K = 128
EPB = NS * K
NB = 80
EPAD = NS * NB * K


def _enc_body(x_ref, w_ref, b_ref, o_ref):
    y = jnp.dot(x_ref[...], w_ref[...], preferred_element_type=jnp.float32)
    y = y + b_ref[...]
    for c in range(C):
        o_ref[c] = y[:, c * CW:(c + 1) * CW]


def _encoder(x_p, W_enc, b_enc):
    return pl.pallas_call(
        _enc_body,
        grid=(NBN,),
        in_specs=[
            pl.BlockSpec((TN, DIN), lambda i: (i, 0)),
            pl.BlockSpec((DIN, H), lambda i: (0, 0)),
            pl.BlockSpec((1, H), lambda i: (0, 0)),
        ],
        out_specs=pl.BlockSpec((C, TN, CW), lambda i: (0, i, 0)),
        out_shape=jax.ShapeDtypeStruct((C, NPAD, CW), jnp.float32),
    )(x_p, W_enc, b_enc.reshape(1, H))


def _mlp_body(h_ref, a_ref, w1_ref, b1_ref, w2_ref, b2_ref, o_ref, *, relu_out):
    u = (h_ref[...] + a_ref[...]).astype(jnp.bfloat16)
    w1 = w1_ref[...].astype(jnp.bfloat16)
    t = jnp.dot(u[0], w1[0:CW, :], preferred_element_type=jnp.float32)
    for c in range(1, C):
        t += jnp.dot(u[c], w1[c * CW:(c + 1) * CW, :],
                     preferred_element_type=jnp.float32)
    t = jnp.maximum(t + b1_ref[...], 0.0).astype(jnp.bfloat16)
    w2 = w2_ref[...].astype(jnp.bfloat16)
    for c in range(C):
        o = jnp.dot(t, w2[:, c * CW:(c + 1) * CW],
                    preferred_element_type=jnp.float32)
        o = o + b2_ref[:, c * CW:(c + 1) * CW]
        if relu_out:
            o = jnp.maximum(o, 0.0)
        o_ref[c] = o


def _gin_mlp(h_ch, agg_ch, W1, b1, W2, b2, relu_out):
    return pl.pallas_call(
        functools.partial(_mlp_body, relu_out=relu_out),
        grid=(NBN,),
        in_specs=[
            pl.BlockSpec((C, TN, CW), lambda i: (0, i, 0)),
            pl.BlockSpec((C, TN, CW), lambda i: (0, i, 0)),
            pl.BlockSpec((H, H), lambda i: (0, 0)),
            pl.BlockSpec((1, H), lambda i: (0, 0)),
            pl.BlockSpec((H, H), lambda i: (0, 0)),
            pl.BlockSpec((1, H), lambda i: (0, 0)),
        ],
        out_specs=pl.BlockSpec((C, TN, CW), lambda i: (0, i, 0)),
        out_shape=jax.ShapeDtypeStruct((C, NPAD, CW), jnp.float32),
    )(h_ch, agg_ch, W1, b1.reshape(1, H), W2, b2.reshape(1, H))


GBK = 80
NSTEP = (NB * K) // GBK


def _agg_body(h_hbm, src_hbm, dst_hbm, z_hbm, out_hbm,
              srcv, dstv, buf, acc, gsem, ssem):
    cid = lax.axis_index("c")
    sid = lax.axis_index("s")
    pltpu.sync_copy(src_hbm.at[sid], srcv)
    pltpu.sync_copy(dst_hbm.at[sid], dstv)
    for ci in range(NC):
        ch = cid * 2 + ci
        pltpu.sync_copy(z_hbm, acc.at[pl.ds(sid * ROWS_PER_TILE, ROWS_PER_TILE)])
        plsc.subcore_barrier()

        pltpu.async_copy(h_hbm.at[ch].at[srcv.at[pl.ds(0, GBK)]],
                         buf.at[pl.ds(0, GBK)], gsem)

        def step(j, carry):
            slot = lax.rem(j, 2) * GBK
            pltpu.make_async_copy(z_hbm.at[pl.ds(0, GBK)],
                                  buf.at[pl.ds(0, GBK)], gsem).wait()
            pltpu.async_copy(buf.at[pl.ds(slot, GBK)],
                             acc.at[dstv.at[pl.ds(j * GBK, GBK)]], ssem,
                             add=True)

            @pl.when(j > 0)
            def _drain_prev_scatter():
                pltpu.make_async_copy(z_hbm.at[pl.ds(0, GBK)],
                                      buf.at[pl.ds(GBK, GBK)], ssem).wait()

            nxt = j + 1

            @pl.when(nxt < NSTEP)
            def _refill():
                pltpu.async_copy(
                    h_hbm.at[ch].at[srcv.at[pl.ds(nxt * GBK, GBK)]],
                    buf.at[pl.ds(lax.rem(nxt, 2) * GBK, GBK)], gsem)

            return carry

        lax.fori_loop(0, NSTEP, step, 0)
        pltpu.make_async_copy(z_hbm.at[pl.ds(0, GBK)],
                              buf.at[pl.ds(0, GBK)], ssem).wait()
        plsc.subcore_barrier()
        pltpu.sync_copy(
            acc.at[pl.ds(sid * ROWS_PER_TILE, ROWS_PER_TILE)],
            out_hbm.at[ch, pl.ds(sid * ROWS_PER_TILE, ROWS_PER_TILE)])


@functools.lru_cache(maxsize=None)
def _make_sc_aggregate():
    mesh = plsc.VectorSubcoreMesh(
        core_axis_name="c", subcore_axis_name="s",
        num_cores=NC, num_subcores=NS)
    return pl.kernel(
        _agg_body,
        out_type=jax.ShapeDtypeStruct((C, NPAD, CW), jnp.float32),
        mesh=mesh,
        scratch_types=[
            pltpu.VMEM((NB * K,), jnp.int32),
            pltpu.VMEM((NB * K,), jnp.int32),
            pltpu.VMEM((2 * GBK, CW), jnp.float32),
            pltpu.VMEM_SHARED((NPAD, CW), jnp.float32),
            pltpu.SemaphoreType.DMA,
            pltpu.SemaphoreType.DMA,
        ],
    )


def _head_body(hg_ref, hc_ref, b_ref, wl_ref, bl_ref, wc_ref, bc_ref,
               o_ref, pg_acc, pc_acc):
    i = pl.program_id(0)

    @pl.when(i == 0)
    def _init():
        pg_acc[...] = jnp.zeros_like(pg_acc)
        pc_acc[...] = jnp.zeros_like(pc_acc)

    bt = b_ref[0]
    gid = lax.broadcasted_iota(jnp.int32, (G, TN), 0)
    oh = jnp.where(bt == gid, 1.0, 0.0)
    for c in range(C):
        pg_acc[:, c * CW:(c + 1) * CW] += jnp.dot(
            oh, hg_ref[c], preferred_element_type=jnp.float32)
        pc_acc[:, c * CW:(c + 1) * CW] += jnp.dot(
            oh, hc_ref[c], preferred_element_type=jnp.float32)

    @pl.when(i == NBN - 1)
    def _final():
        z = jnp.dot(pg_acc[...], wl_ref[0:H, :],
                    preferred_element_type=jnp.float32)
        z += jnp.dot(pc_acc[...], wl_ref[H:2 * H, :],
                     preferred_element_type=jnp.float32)
        z = jnp.maximum(z + bl_ref[...], 0.0)
        o_ref[...] = jnp.dot(z, wc_ref[...],
                             preferred_element_type=jnp.float32) + bc_ref[...]


def _pool_head(hg_ch, hc_ch, batch_r, W_lin, b_lin, W_clf, b_clf):
    return pl.pallas_call(
        _head_body,
        grid=(NBN,),
        in_specs=[
            pl.BlockSpec((C, TN, CW), lambda i: (0, i, 0)),
            pl.BlockSpec((C, TN, CW), lambda i: (0, i, 0)),
            pl.BlockSpec((1, 1, TN), lambda i: (i, 0, 0)),
            pl.BlockSpec((2 * H, H), lambda i: (0, 0)),
            pl.BlockSpec((1, H), lambda i: (0, 0)),
            pl.BlockSpec((H, OUT), lambda i: (0, 0)),
            pl.BlockSpec((1, OUT), lambda i: (0, 0)),
        ],
        out_specs=pl.BlockSpec((G, OUT), lambda i: (0, 0)),
        out_shape=jax.ShapeDtypeStruct((G, OUT), jnp.float32),
        scratch_shapes=[
            pltpu.VMEM((G, H), jnp.float32),
            pltpu.VMEM((G, H), jnp.float32),
        ],
    )(hg_ch, hc_ch, batch_r, W_lin, b_lin.reshape(1, H),
      W_clf, b_clf.reshape(1, OUT))


def kernel(x, edge_index, batch, W_enc, b_enc,
           g0_W1, g0_b1, g0_W2, g0_b2, g1_W1, g1_b1, g1_W2, g1_b2,
           c0_W1, c0_b1, c0_W2, c0_b2, c1_W1, c1_b1, c1_W2, c1_b2,
           W_lin, b_lin, W_clf, b_clf):
    x_p = jnp.pad(x, ((0, NPAD - N), (0, 0)))
    src = jnp.concatenate(
        [edge_index[0], jnp.zeros((EPAD - E,), jnp.int32)]).reshape(NS, NB * K)
    dst = jnp.concatenate(
        [edge_index[1], jnp.full((EPAD - E,), N, jnp.int32)]).reshape(NS, NB * K)
    batch_r = jnp.concatenate(
        [batch, jnp.full((NPAD - N,), G, jnp.int32)]).reshape(NBN, 1, TN)
    zrows = jnp.zeros((ROWS_PER_TILE, CW), jnp.float32)

    h = _encoder(x_p, W_enc, b_enc)
    layers = [(g0_W1, g0_b1, g0_W2, g0_b2, True),
              (g1_W1, g1_b1, g1_W2, g1_b2, False),
              (c0_W1, c0_b1, c0_W2, c0_b2, True),
              (c1_W1, c1_b1, c1_W2, c1_b2, False)]
    hg = None
    for li, (W1, b1, W2, b2, relu_out) in enumerate(layers):
        agg = _make_sc_aggregate()(h, src, dst, zrows)
        h = _gin_mlp(h, agg, W1, b1, W2, b2, relu_out)
        if li == 1:
            hg = h
    return _pool_head(hg, h, batch_r, W_lin, b_lin, W_clf, b_clf)


# speedup vs baseline: 1.0321x; 1.0321x over previous
import functools

import jax
import jax.numpy as jnp
from jax import lax
from jax.experimental import pallas as pl
from jax.experimental.pallas import tpu as pltpu
from jax.experimental.pallas import tpu_sc as plsc

N = 10000
E = 160000
DIN = 256
H = 512
OUT = 128
G = 64

NC = 2
NS = 16
CW = 128
C = H // CW

NPAD = 10240
TN = 1024
NBN = NPAD // TN
ROWS_PER_TILE = NPAD // NS

K = 128
EPB = NS * K
NB = 80
EPAD = NS * NB * K


def _enc_body(x_ref, w_ref, b_ref, o_ref):
    y = jnp.dot(x_ref[...], w_ref[...], preferred_element_type=jnp.float32)
    y = y + b_ref[...]
    for c in range(C):
        o_ref[c] = y[:, c * CW:(c + 1) * CW]


def _encoder(x_p, W_enc, b_enc):
    return pl.pallas_call(
        _enc_body,
        grid=(NBN,),
        in_specs=[
            pl.BlockSpec((TN, DIN), lambda i: (i, 0)),
            pl.BlockSpec((DIN, H), lambda i: (0, 0)),
            pl.BlockSpec((1, H), lambda i: (0, 0)),
        ],
        out_specs=pl.BlockSpec((C, TN, CW), lambda i: (0, i, 0)),
        out_shape=jax.ShapeDtypeStruct((C, NPAD, CW), jnp.float32),
    )(x_p, W_enc, b_enc.reshape(1, H))


def _mlp_p1_body(h_ref, w1_ref, b1_ref, o_ref):
    h = h_ref[...].astype(jnp.bfloat16)
    w1 = w1_ref[...].astype(jnp.bfloat16)
    t = jnp.dot(h[0], w1[0:CW, :], preferred_element_type=jnp.float32)
    for c in range(1, C):
        t += jnp.dot(h[c], w1[c * CW:(c + 1) * CW, :],
                     preferred_element_type=jnp.float32)
    o_ref[...] = t + b1_ref[...]


def _mlp_p1(h_ch, W1, b1):
    return pl.pallas_call(
        _mlp_p1_body,
        grid=(NBN,),
        in_specs=[
            pl.BlockSpec((C, TN, CW), lambda i: (0, i, 0)),
            pl.BlockSpec((H, H), lambda i: (0, 0)),
            pl.BlockSpec((1, H), lambda i: (0, 0)),
        ],
        out_specs=pl.BlockSpec((TN, H), lambda i: (i, 0)),
        out_shape=jax.ShapeDtypeStruct((NPAD, H), jnp.float32),
    )(h_ch, W1, b1.reshape(1, H))


def _mlp_p2_body(p_ref, a_ref, w1_ref, w2_ref, b2_ref, o_ref, *, relu_out):
    a = a_ref[...].astype(jnp.bfloat16)
    w1 = w1_ref[...].astype(jnp.bfloat16)
    t = jnp.dot(a[0], w1[0:CW, :], preferred_element_type=jnp.float32)
    for c in range(1, C):
        t += jnp.dot(a[c], w1[c * CW:(c + 1) * CW, :],
                     preferred_element_type=jnp.float32)
    t = jnp.maximum(t + p_ref[...], 0.0).astype(jnp.bfloat16)
    w2 = w2_ref[...].astype(jnp.bfloat16)
    for c in range(C):
        o = jnp.dot(t, w2[:, c * CW:(c + 1) * CW],
                    preferred_element_type=jnp.float32)
        o = o + b2_ref[:, c * CW:(c + 1) * CW]
        if relu_out:
            o = jnp.maximum(o, 0.0)
        o_ref[c] = o


def _mlp_p2(P, agg_ch, W1, W2, b2, relu_out):
    return pl.pallas_call(
        functools.partial(_mlp_p2_body, relu_out=relu_out),
        grid=(NBN,),
        in_specs=[
            pl.BlockSpec((TN, H), lambda i: (i, 0)),
            pl.BlockSpec((C, TN, CW), lambda i: (0, i, 0)),
            pl.BlockSpec((H, H), lambda i: (0, 0)),
            pl.BlockSpec((H, H), lambda i: (0, 0)),
            pl.BlockSpec((1, H), lambda i: (0, 0)),
        ],
        out_specs=pl.BlockSpec((C, TN, CW), lambda i: (0, i, 0)),
        out_shape=jax.ShapeDtypeStruct((C, NPAD, CW), jnp.float32),
    )(P, agg_ch, W1, W2, b2.reshape(1, H))


GBK = 80
NSTEP = (NB * K) // GBK


def _agg_body(h_hbm, src_hbm, dst_hbm, z_hbm, out_hbm,
              srcv, dstv, buf, acc, gsem, ssem):
    cid = lax.axis_index("c")
    sid = lax.axis_index("s")
    pltpu.sync_copy(src_hbm.at[sid], srcv)
    pltpu.sync_copy(dst_hbm.at[sid], dstv)
    for ci in range(NC):
        ch = cid * 2 + ci
        pltpu.sync_copy(z_hbm, acc.at[pl.ds(sid * ROWS_PER_TILE, ROWS_PER_TILE)])
        plsc.subcore_barrier()

        pltpu.async_copy(h_hbm.at[ch].at[srcv.at[pl.ds(0, GBK)]],
                         buf.at[pl.ds(0, GBK)], gsem)

        def step(j, carry):
            slot = lax.rem(j, 2) * GBK
            pltpu.make_async_copy(z_hbm.at[pl.ds(0, GBK)],
                                  buf.at[pl.ds(0, GBK)], gsem).wait()
            pltpu.async_copy(buf.at[pl.ds(slot, GBK)],
                             acc.at[dstv.at[pl.ds(j * GBK, GBK)]], ssem,
                             add=True)

            @pl.when(j > 0)
            def _drain_prev_scatter():
                pltpu.make_async_copy(z_hbm.at[pl.ds(0, GBK)],
                                      buf.at[pl.ds(GBK, GBK)], ssem).wait()

            nxt = j + 1

            @pl.when(nxt < NSTEP)
            def _refill():
                pltpu.async_copy(
                    h_hbm.at[ch].at[srcv.at[pl.ds(nxt * GBK, GBK)]],
                    buf.at[pl.ds(lax.rem(nxt, 2) * GBK, GBK)], gsem)

            return carry

        lax.fori_loop(0, NSTEP, step, 0)
        pltpu.make_async_copy(z_hbm.at[pl.ds(0, GBK)],
                              buf.at[pl.ds(0, GBK)], ssem).wait()
        plsc.subcore_barrier()
        pltpu.sync_copy(
            acc.at[pl.ds(sid * ROWS_PER_TILE, ROWS_PER_TILE)],
            out_hbm.at[ch, pl.ds(sid * ROWS_PER_TILE, ROWS_PER_TILE)])


@functools.lru_cache(maxsize=None)
def _make_sc_aggregate():
    mesh = plsc.VectorSubcoreMesh(
        core_axis_name="c", subcore_axis_name="s",
        num_cores=NC, num_subcores=NS)
    return pl.kernel(
        _agg_body,
        out_type=jax.ShapeDtypeStruct((C, NPAD, CW), jnp.float32),
        mesh=mesh,
        scratch_types=[
            pltpu.VMEM((NB * K,), jnp.int32),
            pltpu.VMEM((NB * K,), jnp.int32),
            pltpu.VMEM((2 * GBK, CW), jnp.float32),
            pltpu.VMEM_SHARED((NPAD, CW), jnp.float32),
            pltpu.SemaphoreType.DMA,
            pltpu.SemaphoreType.DMA,
        ],
    )


def _head_body(hg_ref, hc_ref, b_ref, wl_ref, bl_ref, wc_ref, bc_ref,
               o_ref, pg_acc, pc_acc):
    i = pl.program_id(0)

    @pl.when(i == 0)
    def _init():
        pg_acc[...] = jnp.zeros_like(pg_acc)
        pc_acc[...] = jnp.zeros_like(pc_acc)

    bt = b_ref[0]
    gid = lax.broadcasted_iota(jnp.int32, (G, TN), 0)
    oh = jnp.where(bt == gid, 1.0, 0.0)
    for c in range(C):
        pg_acc[:, c * CW:(c + 1) * CW] += jnp.dot(
            oh, hg_ref[c], preferred_element_type=jnp.float32)
        pc_acc[:, c * CW:(c + 1) * CW] += jnp.dot(
            oh, hc_ref[c], preferred_element_type=jnp.float32)

    @pl.when(i == NBN - 1)
    def _final():
        z = jnp.dot(pg_acc[...], wl_ref[0:H, :],
                    preferred_element_type=jnp.float32)
        z += jnp.dot(pc_acc[...], wl_ref[H:2 * H, :],
                     preferred_element_type=jnp.float32)
        z = jnp.maximum(z + bl_ref[...], 0.0)
        o_ref[...] = jnp.dot(z, wc_ref[...],
                             preferred_element_type=jnp.float32) + bc_ref[...]


def _pool_head(hg_ch, hc_ch, batch_r, W_lin, b_lin, W_clf, b_clf):
    return pl.pallas_call(
        _head_body,
        grid=(NBN,),
        in_specs=[
            pl.BlockSpec((C, TN, CW), lambda i: (0, i, 0)),
            pl.BlockSpec((C, TN, CW), lambda i: (0, i, 0)),
            pl.BlockSpec((1, 1, TN), lambda i: (i, 0, 0)),
            pl.BlockSpec((2 * H, H), lambda i: (0, 0)),
            pl.BlockSpec((1, H), lambda i: (0, 0)),
            pl.BlockSpec((H, OUT), lambda i: (0, 0)),
            pl.BlockSpec((1, OUT), lambda i: (0, 0)),
        ],
        out_specs=pl.BlockSpec((G, OUT), lambda i: (0, 0)),
        out_shape=jax.ShapeDtypeStruct((G, OUT), jnp.float32),
        scratch_shapes=[
            pltpu.VMEM((G, H), jnp.float32),
            pltpu.VMEM((G, H), jnp.float32),
        ],
    )(hg_ch, hc_ch, batch_r, W_lin, b_lin.reshape(1, H),
      W_clf, b_clf.reshape(1, OUT))


def kernel(x, edge_index, batch, W_enc, b_enc,
           g0_W1, g0_b1, g0_W2, g0_b2, g1_W1, g1_b1, g1_W2, g1_b2,
           c0_W1, c0_b1, c0_W2, c0_b2, c1_W1, c1_b1, c1_W2, c1_b2,
           W_lin, b_lin, W_clf, b_clf):
    x_p = jnp.pad(x, ((0, NPAD - N), (0, 0)))
    src = jnp.concatenate(
        [edge_index[0], jnp.zeros((EPAD - E,), jnp.int32)]).reshape(NS, NB * K)
    dst = jnp.concatenate(
        [edge_index[1], jnp.full((EPAD - E,), N, jnp.int32)]).reshape(NS, NB * K)
    batch_r = jnp.concatenate(
        [batch, jnp.full((NPAD - N,), G, jnp.int32)]).reshape(NBN, 1, TN)
    zrows = jnp.zeros((ROWS_PER_TILE, CW), jnp.float32)

    h = _encoder(x_p, W_enc, b_enc)
    layers = [(g0_W1, g0_b1, g0_W2, g0_b2, True),
              (g1_W1, g1_b1, g1_W2, g1_b2, False),
              (c0_W1, c0_b1, c0_W2, c0_b2, True),
              (c1_W1, c1_b1, c1_W2, c1_b2, False)]
    hg = None
    for li, (W1, b1, W2, b2, relu_out) in enumerate(layers):
        agg = _make_sc_aggregate()(h, src, dst, zrows)
        P = _mlp_p1(h, W1, b1)
        h = _mlp_p2(P, agg, W1, W2, b2, relu_out)
        if li == 1:
            hg = h
    return _pool_head(hg, h, batch_r, W_lin, b_lin, W_clf, b_clf)
